# baseline (speedup 1.0000x reference)
import functools

import jax
import jax.numpy as jnp
from jax import lax
from jax.experimental import pallas as pl
from jax.experimental.pallas import tpu as pltpu
from jax.experimental.pallas import tpu_sc as plsc

N_NODES = 10000
N_EDGES = 320000
D = 128
N_GRAPHS = 128
N_CLASSES = 32

CHUNK = 128
N_TILES = 32
SUBCORES = 16
BURSTS_PER_TILE = 80
GROUP = 8
GROUPS = BURSTS_PER_TILE // GROUP
EDGE_ROWS = BURSTS_PER_TILE * N_TILES + 2 * GROUP
ACC_ROWS = 10128
ZERO_ROWS_PER_TILE = 632
ROWS_PER_TILE = 624
ROWS_TAIL = N_NODES - ROWS_PER_TILE * SUBCORES

_sc_mesh = plsc.VectorSubcoreMesh(core_axis_name="c", subcore_axis_name="s")


@functools.partial(
    pl.kernel,
    mesh=_sc_mesh,
    out_type=jax.ShapeDtypeStruct((2, N_NODES, D), jnp.float32),
    scratch_types=(
        [pltpu.VMEM((CHUNK,), jnp.int32) for _ in range(2)]
        + [pltpu.VMEM((CHUNK, D), jnp.float32)]
        + [pltpu.VMEM_SHARED((ACC_ROWS, D), jnp.float32)]
        + [pltpu.SemaphoreType.DMA]
    ),
)
def _edge_scatter_add(x_hbm, src_hbm, dst_hbm, out_hbm,
                      src_v, dst_v, r0, acc_sh, sem):
    c = lax.axis_index("c")
    s = lax.axis_index("s")
    wid = s * 2 + c

    def _zero_row(r, _):
        def _zero_lane(k, _):
            r0[r, pl.ds(k * 16, 16)] = jnp.zeros((16,), jnp.float32)
            return 0
        return lax.fori_loop(0, D // 16, _zero_lane, 0)
    lax.fori_loop(0, CHUNK, _zero_row, 0)

    zbase = s * ZERO_ROWS_PER_TILE
    for j in range(ZERO_ROWS_PER_TILE // CHUNK):
        pltpu.sync_copy(r0, acc_sh.at[pl.ds(zbase + j * CHUNK, CHUNK)])
    zrem = ZERO_ROWS_PER_TILE % CHUNK
    if zrem:
        pltpu.sync_copy(
            r0.at[pl.ds(0, zrem)],
            acc_sh.at[pl.ds(zbase + (ZERO_ROWS_PER_TILE // CHUNK) * CHUNK,
                            zrem)])
    ztail = ACC_ROWS - ZERO_ROWS_PER_TILE * SUBCORES
    if ztail:
        @pl.when(s == 0)
        def _():
            pltpu.sync_copy(
                r0.at[pl.ds(0, ztail)],
                acc_sh.at[pl.ds(ZERO_ROWS_PER_TILE * SUBCORES, ztail)])

    base = s * ROWS_PER_TILE

    plsc.subcore_barrier()

    n_mine = jnp.int32(BURSTS_PER_TILE)
    b_start = wid * BURSTS_PER_TILE

    def _burst(i, _):
        e0 = b_start + i
        pltpu.sync_copy(src_hbm.at[e0], src_v)
        pltpu.sync_copy(dst_hbm.at[e0], dst_v)
        pltpu.async_copy(x_hbm.at[src_v], r0, sem).wait()
        pltpu.sync_copy(r0, acc_sh.at[dst_v], add=True)
        return 0
    lax.fori_loop(0, n_mine, _burst, 0)

    plsc.subcore_barrier()
    pltpu.sync_copy(acc_sh.at[pl.ds(base, ROWS_PER_TILE)],
                    out_hbm.at[c, pl.ds(base, ROWS_PER_TILE)])

    @pl.when(s == 0)
    def _():
        pltpu.sync_copy(
            acc_sh.at[pl.ds(ROWS_PER_TILE * SUBCORES, ROWS_TAIL)],
            out_hbm.at[c, pl.ds(ROWS_PER_TILE * SUBCORES, ROWS_TAIL)])


ROWS_B = 1000


def _mlp_body(x_ref, a0_ref, a1_ref, wa_ref, ba_ref, wb_ref, bb_ref, o_ref):
    h = x_ref[...] + a0_ref[...] + a1_ref[...]
    h = jnp.dot(h, wa_ref[...], preferred_element_type=jnp.float32) + ba_ref[...]
    h = jnp.maximum(h, 0.0)
    h = jnp.dot(h, wb_ref[...], preferred_element_type=jnp.float32) + bb_ref[...]
    o_ref[...] = jnp.maximum(h, 0.0)


def _mlp(x, a0, a1, wa, ba, wb, bb):
    row_spec = pl.BlockSpec((ROWS_B, D), lambda i: (i, 0))
    w_spec = pl.BlockSpec((D, D), lambda i: (0, 0))
    b_spec = pl.BlockSpec((1, D), lambda i: (0, 0))
    return pl.pallas_call(
        _mlp_body,
        grid=(N_NODES // ROWS_B,),
        in_specs=[row_spec, row_spec, row_spec, w_spec, b_spec, w_spec, b_spec],
        out_specs=row_spec,
        out_shape=jax.ShapeDtypeStruct((N_NODES, D), jnp.float32),
    )(x, a0, a1, wa, ba.reshape(1, D), wb, bb.reshape(1, D))


def _mlp_pool_body(x_ref, a0_ref, a1_ref, wa_ref, ba_ref, wb_ref, bb_ref,
                   batch_ref, o_ref):
    h = x_ref[...] + a0_ref[...] + a1_ref[...]
    h = jnp.dot(h, wa_ref[...], preferred_element_type=jnp.float32) + ba_ref[...]
    h = jnp.maximum(h, 0.0)
    h = jnp.dot(h, wb_ref[...], preferred_element_type=jnp.float32) + bb_ref[...]
    h = jnp.maximum(h, 0.0)
    onehot = (batch_ref[...] == lax.broadcasted_iota(
        jnp.int32, (ROWS_B, N_GRAPHS), 1)).astype(jnp.float32)
    part = lax.dot_general(onehot, h, (((0,), (0,)), ((), ())),
                           preferred_element_type=jnp.float32)

    @pl.when(pl.program_id(0) == 0)
    def _():
        o_ref[...] = part

    @pl.when(pl.program_id(0) > 0)
    def _():
        o_ref[...] += part


def _mlp_pool(x, a0, a1, wa, ba, wb, bb, batch2):
    row_spec = pl.BlockSpec((ROWS_B, D), lambda i: (i, 0))
    w_spec = pl.BlockSpec((D, D), lambda i: (0, 0))
    b_spec = pl.BlockSpec((1, D), lambda i: (0, 0))
    return pl.pallas_call(
        _mlp_pool_body,
        grid=(N_NODES // ROWS_B,),
        in_specs=[row_spec, row_spec, row_spec, w_spec, b_spec, w_spec, b_spec,
                  pl.BlockSpec((ROWS_B, 1), lambda i: (i, 0))],
        out_specs=pl.BlockSpec((N_GRAPHS, N_GRAPHS), lambda i: (0, 0)),
        out_shape=jax.ShapeDtypeStruct((N_GRAPHS, N_GRAPHS), jnp.float32),
    )(x, a0, a1, wa, ba.reshape(1, D), wb, bb.reshape(1, D), batch2)


def _head_body(p_ref, w1_ref, b1_ref, w2_ref, b2_ref, o_ref):
    h = jnp.dot(p_ref[...], w1_ref[...], preferred_element_type=jnp.float32)
    h = jnp.maximum(h + b1_ref[...], 0.0)
    z = jnp.dot(h, w2_ref[...], preferred_element_type=jnp.float32) + b2_ref[...]
    m = jnp.max(z, axis=1, keepdims=True)
    e = jnp.exp(z - m)
    o_ref[...] = z - m - jnp.log(jnp.sum(e, axis=1, keepdims=True))


def _head(pooled, w1, b1, w2, b2):
    return pl.pallas_call(
        _head_body,
        out_shape=jax.ShapeDtypeStruct((N_GRAPHS, N_CLASSES), jnp.float32),
    )(pooled, w1, b1.reshape(1, D), w2, b2.reshape(1, N_CLASSES))


def kernel(x, edge_index, batch, W1a, b1a, W1b, b1b, W2a, b2a, W2b, b2b,
           Wl1, bl1, Wl2, bl2):
    n_pad = EDGE_ROWS * CHUNK - N_EDGES
    src = jnp.concatenate(
        [edge_index[0].astype(jnp.int32), jnp.zeros((n_pad,), jnp.int32)]
    ).reshape(EDGE_ROWS, CHUNK)
    pad_dst = N_NODES + (jnp.arange(n_pad, dtype=jnp.int32) % (ACC_ROWS - N_NODES))
    dst = jnp.concatenate(
        [edge_index[1].astype(jnp.int32), pad_dst]
    ).reshape(EDGE_ROWS, CHUNK)
    batch2 = batch.astype(jnp.int32).reshape(N_NODES, 1)

    agg1 = _edge_scatter_add(x, src, dst)
    h1 = _mlp(x, agg1[0], agg1[1], W1a, b1a, W1b, b1b)
    agg2 = _edge_scatter_add(h1, src, dst)
    pooled = _mlp_pool(h1, agg2[0], agg2[1], W2a, b2a, W2b, b2b, batch2)
    return _head(pooled, Wl1, bl1, Wl2, bl2)


# speedup vs baseline: 1.0995x; 1.0995x over previous
import functools

import jax
import jax.numpy as jnp
from jax import lax
from jax.experimental import pallas as pl
from jax.experimental.pallas import tpu as pltpu
from jax.experimental.pallas import tpu_sc as plsc

N_NODES = 10000
N_EDGES = 320000
D = 128
N_GRAPHS = 128
N_CLASSES = 32

CHUNK = 128
N_TILES = 32
SUBCORES = 16
BURSTS_PER_TILE = 80
GROUP = 8
GROUPS = BURSTS_PER_TILE // GROUP
EDGE_ROWS = BURSTS_PER_TILE * N_TILES + 2 * GROUP
ACC_ROWS = 10128
ZERO_ROWS_PER_TILE = 632
ROWS_PER_TILE = 624
ROWS_TAIL = N_NODES - ROWS_PER_TILE * SUBCORES

_sc_mesh = plsc.VectorSubcoreMesh(core_axis_name="c", subcore_axis_name="s")


@functools.partial(
    pl.kernel,
    mesh=_sc_mesh,
    out_type=jax.ShapeDtypeStruct((2, N_NODES, D), jnp.float32),
    scratch_types=(
        [pltpu.VMEM((CHUNK,), jnp.int32) for _ in range(2)]
        + [pltpu.VMEM((CHUNK, D), jnp.float32)]
        + [pltpu.VMEM_SHARED((ACC_ROWS, D), jnp.float32)]
        + [pltpu.SemaphoreType.DMA]
    ),
)
def _edge_scatter_add(x_hbm, src_hbm, dst_hbm, out_hbm,
                      src_v, dst_v, r0, acc_sh, sem):
    c = lax.axis_index("c")
    s = lax.axis_index("s")
    wid = s * 2 + c

    def _zero_row(r, _):
        def _zero_lane(k, _):
            r0[r, pl.ds(k * 16, 16)] = jnp.zeros((16,), jnp.float32)
            return 0
        return lax.fori_loop(0, D // 16, _zero_lane, 0)
    lax.fori_loop(0, CHUNK, _zero_row, 0)

    zbase = s * ZERO_ROWS_PER_TILE
    for j in range(ZERO_ROWS_PER_TILE // CHUNK):
        pltpu.sync_copy(r0, acc_sh.at[pl.ds(zbase + j * CHUNK, CHUNK)])
    zrem = ZERO_ROWS_PER_TILE % CHUNK
    if zrem:
        pltpu.sync_copy(
            r0.at[pl.ds(0, zrem)],
            acc_sh.at[pl.ds(zbase + (ZERO_ROWS_PER_TILE // CHUNK) * CHUNK,
                            zrem)])
    ztail = ACC_ROWS - ZERO_ROWS_PER_TILE * SUBCORES
    if ztail:
        @pl.when(s == 0)
        def _():
            pltpu.sync_copy(
                r0.at[pl.ds(0, ztail)],
                acc_sh.at[pl.ds(ZERO_ROWS_PER_TILE * SUBCORES, ztail)])

    base = s * ROWS_PER_TILE

    plsc.subcore_barrier()

    n_mine = jnp.int32(BURSTS_PER_TILE)
    b_start = wid * BURSTS_PER_TILE

    def _burst(i, _):
        e0 = (b_start + i) * CHUNK
        pltpu.sync_copy(src_hbm.at[pl.ds(e0, CHUNK)], src_v)
        pltpu.sync_copy(dst_hbm.at[pl.ds(e0, CHUNK)], dst_v)
        pltpu.async_copy(x_hbm.at[src_v], r0, sem).wait()
        pltpu.sync_copy(r0, acc_sh.at[dst_v], add=True)
        return 0
    lax.fori_loop(0, n_mine, _burst, 0)

    plsc.subcore_barrier()
    pltpu.sync_copy(acc_sh.at[pl.ds(base, ROWS_PER_TILE)],
                    out_hbm.at[c, pl.ds(base, ROWS_PER_TILE)])

    @pl.when(s == 0)
    def _():
        pltpu.sync_copy(
            acc_sh.at[pl.ds(ROWS_PER_TILE * SUBCORES, ROWS_TAIL)],
            out_hbm.at[c, pl.ds(ROWS_PER_TILE * SUBCORES, ROWS_TAIL)])


ROWS_B = 1000


def _mlp_body(x_ref, a0_ref, a1_ref, wa_ref, ba_ref, wb_ref, bb_ref, o_ref):
    h = x_ref[...] + a0_ref[...] + a1_ref[...]
    h = jnp.dot(h, wa_ref[...], preferred_element_type=jnp.float32) + ba_ref[...]
    h = jnp.maximum(h, 0.0)
    h = jnp.dot(h, wb_ref[...], preferred_element_type=jnp.float32) + bb_ref[...]
    o_ref[...] = jnp.maximum(h, 0.0)


def _mlp(x, a0, a1, wa, ba, wb, bb):
    row_spec = pl.BlockSpec((ROWS_B, D), lambda i: (i, 0))
    w_spec = pl.BlockSpec((D, D), lambda i: (0, 0))
    b_spec = pl.BlockSpec((1, D), lambda i: (0, 0))
    return pl.pallas_call(
        _mlp_body,
        grid=(N_NODES // ROWS_B,),
        in_specs=[row_spec, row_spec, row_spec, w_spec, b_spec, w_spec, b_spec],
        out_specs=row_spec,
        out_shape=jax.ShapeDtypeStruct((N_NODES, D), jnp.float32),
    )(x, a0, a1, wa, ba.reshape(1, D), wb, bb.reshape(1, D))


def _mlp_pool_body(x_ref, a0_ref, a1_ref, wa_ref, ba_ref, wb_ref, bb_ref,
                   batch_ref, o_ref):
    h = x_ref[...] + a0_ref[...] + a1_ref[...]
    h = jnp.dot(h, wa_ref[...], preferred_element_type=jnp.float32) + ba_ref[...]
    h = jnp.maximum(h, 0.0)
    h = jnp.dot(h, wb_ref[...], preferred_element_type=jnp.float32) + bb_ref[...]
    h = jnp.maximum(h, 0.0)
    onehot = (batch_ref[...] == lax.broadcasted_iota(
        jnp.int32, (ROWS_B, N_GRAPHS), 1)).astype(jnp.float32)
    part = lax.dot_general(onehot, h, (((0,), (0,)), ((), ())),
                           preferred_element_type=jnp.float32)

    @pl.when(pl.program_id(0) == 0)
    def _():
        o_ref[...] = part

    @pl.when(pl.program_id(0) > 0)
    def _():
        o_ref[...] += part


def _mlp_pool(x, a0, a1, wa, ba, wb, bb, batch2):
    row_spec = pl.BlockSpec((ROWS_B, D), lambda i: (i, 0))
    w_spec = pl.BlockSpec((D, D), lambda i: (0, 0))
    b_spec = pl.BlockSpec((1, D), lambda i: (0, 0))
    return pl.pallas_call(
        _mlp_pool_body,
        grid=(N_NODES // ROWS_B,),
        in_specs=[row_spec, row_spec, row_spec, w_spec, b_spec, w_spec, b_spec,
                  pl.BlockSpec((ROWS_B, 1), lambda i: (i, 0))],
        out_specs=pl.BlockSpec((N_GRAPHS, N_GRAPHS), lambda i: (0, 0)),
        out_shape=jax.ShapeDtypeStruct((N_GRAPHS, N_GRAPHS), jnp.float32),
    )(x, a0, a1, wa, ba.reshape(1, D), wb, bb.reshape(1, D), batch2)


def _head_body(p_ref, w1_ref, b1_ref, w2_ref, b2_ref, o_ref):
    h = jnp.dot(p_ref[...], w1_ref[...], preferred_element_type=jnp.float32)
    h = jnp.maximum(h + b1_ref[...], 0.0)
    z = jnp.dot(h, w2_ref[...], preferred_element_type=jnp.float32) + b2_ref[...]
    m = jnp.max(z, axis=1, keepdims=True)
    e = jnp.exp(z - m)
    o_ref[...] = z - m - jnp.log(jnp.sum(e, axis=1, keepdims=True))


def _head(pooled, w1, b1, w2, b2):
    return pl.pallas_call(
        _head_body,
        out_shape=jax.ShapeDtypeStruct((N_GRAPHS, N_CLASSES), jnp.float32),
    )(pooled, w1, b1.reshape(1, D), w2, b2.reshape(1, N_CLASSES))


def kernel(x, edge_index, batch, W1a, b1a, W1b, b1b, W2a, b2a, W2b, b2b,
           Wl1, bl1, Wl2, bl2):
    n_pad = EDGE_ROWS * CHUNK - N_EDGES
    src = jnp.concatenate(
        [edge_index[0].astype(jnp.int32), jnp.zeros((n_pad,), jnp.int32)])
    pad_dst = N_NODES + (jnp.arange(n_pad, dtype=jnp.int32) % (ACC_ROWS - N_NODES))
    dst = jnp.concatenate([edge_index[1].astype(jnp.int32), pad_dst])
    batch2 = batch.astype(jnp.int32).reshape(N_NODES, 1)

    agg1 = _edge_scatter_add(x, src, dst)
    h1 = _mlp(x, agg1[0], agg1[1], W1a, b1a, W1b, b1b)
    agg2 = _edge_scatter_add(h1, src, dst)
    pooled = _mlp_pool(h1, agg2[0], agg2[1], W2a, b2a, W2b, b2b, batch2)
    return _head(pooled, Wl1, bl1, Wl2, bl2)


# speedup vs baseline: 2.2885x; 2.0814x over previous
import functools

import jax
import jax.numpy as jnp
from jax import lax
from jax.experimental import pallas as pl
from jax.experimental.pallas import tpu as pltpu
from jax.experimental.pallas import tpu_sc as plsc

N_NODES = 10000
N_EDGES = 320000
D = 128
N_GRAPHS = 128
N_CLASSES = 32

CHUNK = 128
N_TILES = 32
SUBCORES = 16
BURSTS_PER_TILE = 80
GROUP = 8
GROUPS = BURSTS_PER_TILE // GROUP
EDGE_ROWS = BURSTS_PER_TILE * N_TILES + 2 * GROUP
ACC_ROWS = 10128
ZERO_ROWS_PER_TILE = 632
ROWS_PER_TILE = 624
ROWS_TAIL = N_NODES - ROWS_PER_TILE * SUBCORES

_sc_mesh = plsc.VectorSubcoreMesh(core_axis_name="c", subcore_axis_name="s")


@functools.partial(
    pl.kernel,
    mesh=_sc_mesh,
    out_type=jax.ShapeDtypeStruct((2, N_NODES, D), jnp.float32),
    scratch_types=(
        [pltpu.VMEM((CHUNK,), jnp.int32) for _ in range(2)]
        + [pltpu.VMEM((CHUNK, D), jnp.float32)]
        + [pltpu.VMEM_SHARED((ACC_ROWS, D), jnp.float32)]
        + [pltpu.SemaphoreType.DMA]
    ),
)
def _edge_scatter_add(x_hbm, src_hbm, dst_hbm, out_hbm,
                      src_v, dst_v, r0, acc_sh, sem):
    c = lax.axis_index("c")
    s = lax.axis_index("s")
    wid = s * 2 + c

    def _zero_row(r, _):
        def _zero_lane(k, _):
            r0[r, pl.ds(k * 16, 16)] = jnp.zeros((16,), jnp.float32)
            return 0
        return lax.fori_loop(0, D // 16, _zero_lane, 0)
    lax.fori_loop(0, CHUNK, _zero_row, 0)

    zbase = s * ZERO_ROWS_PER_TILE
    for j in range(ZERO_ROWS_PER_TILE // CHUNK):
        pltpu.sync_copy(r0, acc_sh.at[pl.ds(zbase + j * CHUNK, CHUNK)])
    zrem = ZERO_ROWS_PER_TILE % CHUNK
    if zrem:
        pltpu.sync_copy(
            r0.at[pl.ds(0, zrem)],
            acc_sh.at[pl.ds(zbase + (ZERO_ROWS_PER_TILE // CHUNK) * CHUNK,
                            zrem)])
    ztail = ACC_ROWS - ZERO_ROWS_PER_TILE * SUBCORES
    if ztail:
        @pl.when(s == 0)
        def _():
            pltpu.sync_copy(
                r0.at[pl.ds(0, ztail)],
                acc_sh.at[pl.ds(ZERO_ROWS_PER_TILE * SUBCORES, ztail)])

    base = s * ROWS_PER_TILE

    plsc.subcore_barrier()

    n_mine = jnp.int32(BURSTS_PER_TILE)
    b_start = wid * BURSTS_PER_TILE

    def _burst(i, _):
        e0 = (b_start + i) * CHUNK
        pltpu.sync_copy(src_hbm.at[pl.ds(e0, CHUNK)], src_v)
        pltpu.sync_copy(dst_hbm.at[pl.ds(e0, CHUNK)], dst_v)
        pltpu.async_copy(x_hbm.at[src_v], r0, sem).wait()
        pltpu.sync_copy(r0, acc_sh.at[dst_v], add=True)
        return 0
    lax.fori_loop(0, n_mine, _burst, 0)

    plsc.subcore_barrier()
    pltpu.sync_copy(acc_sh.at[pl.ds(base, ROWS_PER_TILE)],
                    out_hbm.at[c, pl.ds(base, ROWS_PER_TILE)])

    @pl.when(s == 0)
    def _():
        pltpu.sync_copy(
            acc_sh.at[pl.ds(ROWS_PER_TILE * SUBCORES, ROWS_TAIL)],
            out_hbm.at[c, pl.ds(ROWS_PER_TILE * SUBCORES, ROWS_TAIL)])


ROWS_B = 1000


def _mlp_body(x_ref, a0_ref, a1_ref, wa_ref, ba_ref, wb_ref, bb_ref, o_ref):
    h = x_ref[...] + a0_ref[...] + a1_ref[...]
    h = jnp.dot(h, wa_ref[...], preferred_element_type=jnp.float32) + ba_ref[...]
    h = jnp.maximum(h, 0.0)
    h = jnp.dot(h, wb_ref[...], preferred_element_type=jnp.float32) + bb_ref[...]
    o_ref[...] = jnp.maximum(h, 0.0)


def _mlp(x, a0, a1, wa, ba, wb, bb):
    row_spec = pl.BlockSpec((ROWS_B, D), lambda i: (i, 0))
    w_spec = pl.BlockSpec((D, D), lambda i: (0, 0))
    b_spec = pl.BlockSpec((1, D), lambda i: (0, 0))
    return pl.pallas_call(
        _mlp_body,
        grid=(N_NODES // ROWS_B,),
        in_specs=[row_spec, row_spec, row_spec, w_spec, b_spec, w_spec, b_spec],
        out_specs=row_spec,
        out_shape=jax.ShapeDtypeStruct((N_NODES, D), jnp.float32),
    )(x, a0, a1, wa, ba.reshape(1, D), wb, bb.reshape(1, D))


def _mlp_pool_body(x_ref, a0_ref, a1_ref, wa_ref, ba_ref, wb_ref, bb_ref,
                   batch_ref, o_ref):
    h = x_ref[...] + a0_ref[...] + a1_ref[...]
    h = jnp.dot(h, wa_ref[...], preferred_element_type=jnp.float32) + ba_ref[...]
    h = jnp.maximum(h, 0.0)
    h = jnp.dot(h, wb_ref[...], preferred_element_type=jnp.float32) + bb_ref[...]
    h = jnp.maximum(h, 0.0)
    onehot = (batch_ref[...] == lax.broadcasted_iota(
        jnp.int32, (ROWS_B, N_GRAPHS), 1)).astype(jnp.float32)
    part = lax.dot_general(onehot, h, (((0,), (0,)), ((), ())),
                           preferred_element_type=jnp.float32)

    @pl.when(pl.program_id(0) == 0)
    def _():
        o_ref[...] = part

    @pl.when(pl.program_id(0) > 0)
    def _():
        o_ref[...] += part


def _mlp_pool(x, a0, a1, wa, ba, wb, bb, batch2):
    row_spec = pl.BlockSpec((ROWS_B, D), lambda i: (i, 0))
    w_spec = pl.BlockSpec((D, D), lambda i: (0, 0))
    b_spec = pl.BlockSpec((1, D), lambda i: (0, 0))
    return pl.pallas_call(
        _mlp_pool_body,
        grid=(N_NODES // ROWS_B,),
        in_specs=[row_spec, row_spec, row_spec, w_spec, b_spec, w_spec, b_spec,
                  pl.BlockSpec((ROWS_B, 1), lambda i: (i, 0))],
        out_specs=pl.BlockSpec((N_GRAPHS, N_GRAPHS), lambda i: (0, 0)),
        out_shape=jax.ShapeDtypeStruct((N_GRAPHS, N_GRAPHS), jnp.float32),
    )(x, a0, a1, wa, ba.reshape(1, D), wb, bb.reshape(1, D), batch2)


def _head_body(p_ref, w1_ref, b1_ref, w2_ref, b2_ref, o_ref):
    h = jnp.dot(p_ref[...], w1_ref[...], preferred_element_type=jnp.float32)
    h = jnp.maximum(h + b1_ref[...], 0.0)
    z = jnp.dot(h, w2_ref[...], preferred_element_type=jnp.float32) + b2_ref[...]
    m = jnp.max(z, axis=1, keepdims=True)
    e = jnp.exp(z - m)
    o_ref[...] = z - m - jnp.log(jnp.sum(e, axis=1, keepdims=True))


def _head(pooled, w1, b1, w2, b2):
    return pl.pallas_call(
        _head_body,
        out_shape=jax.ShapeDtypeStruct((N_GRAPHS, N_CLASSES), jnp.float32),
    )(pooled, w1, b1.reshape(1, D), w2, b2.reshape(1, N_CLASSES))


def kernel(x, edge_index, batch, W1a, b1a, W1b, b1b, W2a, b2a, W2b, b2b,
           Wl1, bl1, Wl2, bl2):
    n_pad = EDGE_ROWS * CHUNK - N_EDGES
    pad_src = jnp.arange(n_pad, dtype=jnp.int32) % N_NODES
    src = jnp.concatenate([edge_index[0].astype(jnp.int32), pad_src])
    pad_dst = N_NODES + (jnp.arange(n_pad, dtype=jnp.int32) % (ACC_ROWS - N_NODES))
    dst = jnp.concatenate([edge_index[1].astype(jnp.int32), pad_dst])
    batch2 = batch.astype(jnp.int32).reshape(N_NODES, 1)

    agg1 = _edge_scatter_add(x, src, dst)
    h1 = _mlp(x, agg1[0], agg1[1], W1a, b1a, W1b, b1b)
    agg2 = _edge_scatter_add(h1, src, dst)
    pooled = _mlp_pool(h1, agg2[0], agg2[1], W2a, b2a, W2b, b2b, batch2)
    return _head(pooled, Wl1, bl1, Wl2, bl2)


# speedup vs baseline: 3.0979x; 1.3537x over previous
import functools

import jax
import jax.numpy as jnp
from jax import lax
from jax.experimental import pallas as pl
from jax.experimental.pallas import tpu as pltpu
from jax.experimental.pallas import tpu_sc as plsc

N_NODES = 10000
N_EDGES = 320000
D = 128
N_GRAPHS = 128
N_CLASSES = 32

CHUNK = 128
N_TILES = 32
SUBCORES = 16
BURSTS_PER_TILE = 80
GROUP = 8
GROUPS = BURSTS_PER_TILE // GROUP
EDGE_ROWS = BURSTS_PER_TILE * N_TILES + 2 * GROUP
ACC_ROWS = 10128
ZERO_ROWS_PER_TILE = 632
ROWS_PER_TILE = 624
ROWS_TAIL = N_NODES - ROWS_PER_TILE * SUBCORES

_sc_mesh = plsc.VectorSubcoreMesh(core_axis_name="c", subcore_axis_name="s")


@functools.partial(
    pl.kernel,
    mesh=_sc_mesh,
    out_type=jax.ShapeDtypeStruct((2, N_NODES, D), jnp.float32),
    scratch_types=(
        [pltpu.VMEM((CHUNK,), jnp.int32) for _ in range(4)]
        + [pltpu.VMEM((CHUNK, D), jnp.float32) for _ in range(2)]
        + [pltpu.VMEM_SHARED((ACC_ROWS, D), jnp.float32)]
        + [pltpu.SemaphoreType.DMA] * 2
    ),
)
def _edge_scatter_add(x_hbm, src_hbm, dst_hbm, out_hbm,
                      src_v, dst_v, srcB_v, dstB_v, r0, r1, acc_sh, sem, semB):
    c = lax.axis_index("c")
    s = lax.axis_index("s")
    wid = s * 2 + c

    def _zero_row(r, _):
        def _zero_lane(k, _):
            r0[r, pl.ds(k * 16, 16)] = jnp.zeros((16,), jnp.float32)
            return 0
        return lax.fori_loop(0, D // 16, _zero_lane, 0)
    lax.fori_loop(0, CHUNK, _zero_row, 0)

    zbase = s * ZERO_ROWS_PER_TILE
    for j in range(ZERO_ROWS_PER_TILE // CHUNK):
        pltpu.sync_copy(r0, acc_sh.at[pl.ds(zbase + j * CHUNK, CHUNK)])
    zrem = ZERO_ROWS_PER_TILE % CHUNK
    if zrem:
        pltpu.sync_copy(
            r0.at[pl.ds(0, zrem)],
            acc_sh.at[pl.ds(zbase + (ZERO_ROWS_PER_TILE // CHUNK) * CHUNK,
                            zrem)])
    ztail = ACC_ROWS - ZERO_ROWS_PER_TILE * SUBCORES
    if ztail:
        @pl.when(s == 0)
        def _():
            pltpu.sync_copy(
                r0.at[pl.ds(0, ztail)],
                acc_sh.at[pl.ds(ZERO_ROWS_PER_TILE * SUBCORES, ztail)])

    base = s * ROWS_PER_TILE

    plsc.subcore_barrier()

    n_mine = jnp.int32(BURSTS_PER_TILE)
    b_start = wid * BURSTS_PER_TILE

    def _pair(i, _):
        eA = (b_start + 2 * i) * CHUNK
        eB = eA + CHUNK
        pltpu.sync_copy(src_hbm.at[pl.ds(eA, CHUNK)], src_v)
        pltpu.sync_copy(dst_hbm.at[pl.ds(eA, CHUNK)], dst_v)
        pltpu.async_copy(x_hbm.at[src_v], r0, sem)
        pltpu.sync_copy(src_hbm.at[pl.ds(eB, CHUNK)], srcB_v)
        pltpu.sync_copy(dst_hbm.at[pl.ds(eB, CHUNK)], dstB_v)
        pltpu.async_copy(x_hbm.at[srcB_v], r1, semB)
        pltpu.make_async_copy(x_hbm.at[pl.ds(0, CHUNK)], r0, sem).wait()
        pltpu.sync_copy(r0, acc_sh.at[dst_v], add=True)
        pltpu.make_async_copy(x_hbm.at[pl.ds(0, CHUNK)], r1, semB).wait()
        pltpu.sync_copy(r1, acc_sh.at[dstB_v], add=True)
        return 0
    lax.fori_loop(0, n_mine // 2, _pair, 0)

    plsc.subcore_barrier()
    pltpu.sync_copy(acc_sh.at[pl.ds(base, ROWS_PER_TILE)],
                    out_hbm.at[c, pl.ds(base, ROWS_PER_TILE)])

    @pl.when(s == 0)
    def _():
        pltpu.sync_copy(
            acc_sh.at[pl.ds(ROWS_PER_TILE * SUBCORES, ROWS_TAIL)],
            out_hbm.at[c, pl.ds(ROWS_PER_TILE * SUBCORES, ROWS_TAIL)])


ROWS_B = 1000


def _mlp_body(x_ref, a0_ref, a1_ref, wa_ref, ba_ref, wb_ref, bb_ref, o_ref):
    h = x_ref[...] + a0_ref[...] + a1_ref[...]
    h = jnp.dot(h, wa_ref[...], preferred_element_type=jnp.float32) + ba_ref[...]
    h = jnp.maximum(h, 0.0)
    h = jnp.dot(h, wb_ref[...], preferred_element_type=jnp.float32) + bb_ref[...]
    o_ref[...] = jnp.maximum(h, 0.0)


def _mlp(x, a0, a1, wa, ba, wb, bb):
    row_spec = pl.BlockSpec((ROWS_B, D), lambda i: (i, 0))
    w_spec = pl.BlockSpec((D, D), lambda i: (0, 0))
    b_spec = pl.BlockSpec((1, D), lambda i: (0, 0))
    return pl.pallas_call(
        _mlp_body,
        grid=(N_NODES // ROWS_B,),
        in_specs=[row_spec, row_spec, row_spec, w_spec, b_spec, w_spec, b_spec],
        out_specs=row_spec,
        out_shape=jax.ShapeDtypeStruct((N_NODES, D), jnp.float32),
    )(x, a0, a1, wa, ba.reshape(1, D), wb, bb.reshape(1, D))


def _mlp_pool_body(x_ref, a0_ref, a1_ref, wa_ref, ba_ref, wb_ref, bb_ref,
                   batch_ref, o_ref):
    h = x_ref[...] + a0_ref[...] + a1_ref[...]
    h = jnp.dot(h, wa_ref[...], preferred_element_type=jnp.float32) + ba_ref[...]
    h = jnp.maximum(h, 0.0)
    h = jnp.dot(h, wb_ref[...], preferred_element_type=jnp.float32) + bb_ref[...]
    h = jnp.maximum(h, 0.0)
    onehot = (batch_ref[...] == lax.broadcasted_iota(
        jnp.int32, (ROWS_B, N_GRAPHS), 1)).astype(jnp.float32)
    part = lax.dot_general(onehot, h, (((0,), (0,)), ((), ())),
                           preferred_element_type=jnp.float32)

    @pl.when(pl.program_id(0) == 0)
    def _():
        o_ref[...] = part

    @pl.when(pl.program_id(0) > 0)
    def _():
        o_ref[...] += part


def _mlp_pool(x, a0, a1, wa, ba, wb, bb, batch2):
    row_spec = pl.BlockSpec((ROWS_B, D), lambda i: (i, 0))
    w_spec = pl.BlockSpec((D, D), lambda i: (0, 0))
    b_spec = pl.BlockSpec((1, D), lambda i: (0, 0))
    return pl.pallas_call(
        _mlp_pool_body,
        grid=(N_NODES // ROWS_B,),
        in_specs=[row_spec, row_spec, row_spec, w_spec, b_spec, w_spec, b_spec,
                  pl.BlockSpec((ROWS_B, 1), lambda i: (i, 0))],
        out_specs=pl.BlockSpec((N_GRAPHS, N_GRAPHS), lambda i: (0, 0)),
        out_shape=jax.ShapeDtypeStruct((N_GRAPHS, N_GRAPHS), jnp.float32),
    )(x, a0, a1, wa, ba.reshape(1, D), wb, bb.reshape(1, D), batch2)


def _head_body(p_ref, w1_ref, b1_ref, w2_ref, b2_ref, o_ref):
    h = jnp.dot(p_ref[...], w1_ref[...], preferred_element_type=jnp.float32)
    h = jnp.maximum(h + b1_ref[...], 0.0)
    z = jnp.dot(h, w2_ref[...], preferred_element_type=jnp.float32) + b2_ref[...]
    m = jnp.max(z, axis=1, keepdims=True)
    e = jnp.exp(z - m)
    o_ref[...] = z - m - jnp.log(jnp.sum(e, axis=1, keepdims=True))


def _head(pooled, w1, b1, w2, b2):
    return pl.pallas_call(
        _head_body,
        out_shape=jax.ShapeDtypeStruct((N_GRAPHS, N_CLASSES), jnp.float32),
    )(pooled, w1, b1.reshape(1, D), w2, b2.reshape(1, N_CLASSES))


def kernel(x, edge_index, batch, W1a, b1a, W1b, b1b, W2a, b2a, W2b, b2b,
           Wl1, bl1, Wl2, bl2):
    n_pad = EDGE_ROWS * CHUNK - N_EDGES
    pad_src = jnp.arange(n_pad, dtype=jnp.int32) % N_NODES
    src = jnp.concatenate([edge_index[0].astype(jnp.int32), pad_src])
    pad_dst = N_NODES + (jnp.arange(n_pad, dtype=jnp.int32) % (ACC_ROWS - N_NODES))
    dst = jnp.concatenate([edge_index[1].astype(jnp.int32), pad_dst])
    batch2 = batch.astype(jnp.int32).reshape(N_NODES, 1)

    agg1 = _edge_scatter_add(x, src, dst)
    h1 = _mlp(x, agg1[0], agg1[1], W1a, b1a, W1b, b1b)
    agg2 = _edge_scatter_add(h1, src, dst)
    pooled = _mlp_pool(h1, agg2[0], agg2[1], W2a, b2a, W2b, b2b, batch2)
    return _head(pooled, Wl1, bl1, Wl2, bl2)
